# Initial kernel scaffold; baseline (speedup 1.0000x reference)
import jax
import jax.numpy as jnp
from jax.experimental import pallas as pl


def kernel(x, edge_index, W, b):
    raise NotImplementedError("write your pallas kernel here")


# speedup vs baseline: 21.1947x; 21.1947x over previous
import functools

import jax
import jax.numpy as jnp
from jax import lax
from jax.experimental import pallas as pl
from jax.experimental.pallas import tpu as pltpu
from jax.experimental.pallas import tpu_sc as plsc

NC = 2
NS = 16
LANES = 16


def _largest_div(total, hi, step):
    for c in range(hi, step - 1, -step):
        if total % c == 0:
            return c
    return None


def _project(x2, W, b2):
    M, Cin = x2.shape
    Cout = W.shape[1]
    BM = _largest_div(M, 2048, 8) or M

    def body(x_ref, w_ref, b_ref, o_ref):
        o_ref[...] = (
            jnp.dot(x_ref[...], w_ref[...], preferred_element_type=jnp.float32)
            + b_ref[...]
        )

    return pl.pallas_call(
        body,
        grid=(M // BM,),
        in_specs=[
            pl.BlockSpec((BM, Cin), lambda i: (i, 0)),
            pl.BlockSpec((Cin, Cout), lambda i: (0, 0)),
            pl.BlockSpec((1, Cout), lambda i: (0, 0)),
        ],
        out_specs=pl.BlockSpec((BM, Cout), lambda i: (i, 0)),
        out_shape=jax.ShapeDtypeStruct((M, Cout), jnp.float32),
    )(x2, W, b2)


def _aggregate(h, src_flat, dst_flat, B, N, E, C):
    assert B % NC == 0 and E % NS == 0 and N % NS == 0 and C % LANES == 0
    ROUNDS = B // NC
    EPT = E // NS
    CH = _largest_div(EPT, 128, 8)
    assert CH is not None
    NCHUNK = EPT // CH
    RCH = _largest_div(N, 128, 16)
    assert RCH is not None
    NROWCH = N // RCH
    ITER_R = -(-NROWCH // NS)
    NCC = C // LANES
    BUF = max(CH, RCH)

    mesh = plsc.VectorSubcoreMesh(core_axis_name="c", subcore_axis_name="s")

    @functools.partial(
        pl.kernel,
        mesh=mesh,
        out_type=jax.ShapeDtypeStruct((B * N, C), jnp.float32),
        scratch_types=[
            pltpu.VMEM((CH,), jnp.int32),
            pltpu.VMEM((CH,), jnp.int32),
            pltpu.VMEM((BUF, C), jnp.float32),
            pltpu.VMEM((BUF,), jnp.float32),
            pltpu.VMEM_SHARED((N, C), jnp.float32),
            pltpu.VMEM_SHARED((N,), jnp.float32),
            pltpu.SemaphoreType.DMA,
        ],
    )
    def agg(h_hbm, src_hbm, dst_hbm, out_hbm,
            idx_s, idx_d, rowbuf, smallbuf,
            acc_sh, deg_sh, gsem):
        c = lax.axis_index("c")
        s = lax.axis_index("s")

        one16 = jnp.full((LANES,), 1.0, jnp.float32)
        zero16 = jnp.zeros((LANES,), jnp.float32)

        def fill_small(val):
            def fbody(i, _):
                smallbuf[pl.ds(i * LANES, LANES)] = val
                return 0
            lax.fori_loop(0, BUF // LANES, fbody, 0)

        for r in range(ROUNDS):
            g = r * NC + c

            fill_small(zero16)

            def zbody(i, _):
                for cc in range(NCC):
                    rowbuf[i, pl.ds(cc * LANES, LANES)] = zero16
                return 0
            lax.fori_loop(0, BUF, zbody, 0)
            for i in range(ITER_R):
                k = i * NS + s

                @pl.when(k < NROWCH)
                def _():
                    rb = k * RCH
                    pltpu.sync_copy(rowbuf.at[pl.ds(0, RCH)], acc_sh.at[pl.ds(rb, RCH)])
                    pltpu.sync_copy(smallbuf.at[pl.ds(0, RCH)], deg_sh.at[pl.ds(rb, RCH)])
            fill_small(one16)
            plsc.subcore_barrier()

            ebase = g * E + s * EPT

            def chunk(j, _):
                e0 = ebase + j * CH
                pltpu.sync_copy(src_hbm.at[pl.ds(e0, CH)], idx_s)
                pltpu.sync_copy(dst_hbm.at[pl.ds(e0, CH)], idx_d)
                pltpu.async_copy(h_hbm.at[idx_s], rowbuf.at[pl.ds(0, CH)], gsem).wait()
                pltpu.sync_copy(rowbuf.at[pl.ds(0, CH)], acc_sh.at[idx_d], add=True)
                pltpu.sync_copy(smallbuf.at[pl.ds(0, CH)], deg_sh.at[idx_d], add=True)
                return 0
            lax.fori_loop(0, NCHUNK, chunk, 0)
            plsc.subcore_barrier()

            for i in range(ITER_R):
                k = i * NS + s

                @pl.when(k < NROWCH)
                def _():
                    rb = k * RCH
                    pltpu.sync_copy(acc_sh.at[pl.ds(rb, RCH)], rowbuf.at[pl.ds(0, RCH)])
                    pltpu.sync_copy(deg_sh.at[pl.ds(rb, RCH)], smallbuf.at[pl.ds(0, RCH)])

                    def rbody(q, _):
                        dvec = smallbuf[pl.ds(q * LANES, LANES)]
                        rec = one16 / jnp.maximum(dvec, one16)
                        for rr in range(LANES):
                            r2 = q * LANES + rr
                            rec16 = jnp.broadcast_to(rec[rr], (LANES,))
                            for cc in range(NCC):
                                sl = pl.ds(cc * LANES, LANES)
                                rowbuf[r2, sl] = rowbuf[r2, sl] * rec16
                        return 0
                    lax.fori_loop(0, RCH // LANES, rbody, 0)
                    pltpu.sync_copy(rowbuf.at[pl.ds(0, RCH)], out_hbm.at[pl.ds(g * N + rb, RCH)])
            plsc.subcore_barrier()

    return agg(h, src_flat, dst_flat)


def kernel(x, edge_index, W, b):
    B, N, Cin = x.shape
    Cout = W.shape[1]
    E = edge_index.shape[2]

    h = _project(x.reshape(B * N, Cin), W, b.reshape(1, Cout))

    offs = (jnp.arange(B, dtype=jnp.int32) * N)[:, None]
    src = (edge_index[:, 1, :] + offs).reshape(B * E)
    dst = edge_index[:, 0, :].reshape(B * E)

    out = _aggregate(h, src, dst, B=B, N=N, E=E, C=Cout)
    return out.reshape(B, N, Cout)
